# Initial kernel scaffold; baseline (speedup 1.0000x reference)
import jax
import jax.numpy as jnp
from jax.experimental import pallas as pl


def kernel(x, edge_index, orders, dist, W_mp, b_mp, W_out, b_out):
    raise NotImplementedError("write your pallas kernel here")


# speedup vs baseline: 30.1463x; 30.1463x over previous
import jax
import jax.numpy as jnp
from jax import lax
from jax.experimental import pallas as pl
from jax.experimental.pallas import tpu as pltpu
from jax.experimental.pallas import tpu_sc as plsc

N_NODES = 100000
NPAD = 100352
DF = 16
E_ORD = 3200000
IDXW = 128
ORD_ROWS = E_ORD // IDXW
BLK = 8
NBLK = ORD_ROWS // BLK
NC = 2
NS = 16
NW = NC * NS
BLK_HI = -(-NBLK // NW)
N_HI_WORKERS = NBLK - NW * (BLK_HI - 1)
ROWS_PER_SUB = NPAD // NS


def _sc_body(ord_r, dst_r, src_r, dist_r, xp_r, z16_r, z1_r,
             s_out, dsum_out,
             ord_v,
             d0, d1, d2, d3, d4, d5, d6, d7,
             s0, s1, s2, s3, s4, s5, s6, s7,
             e0, e1, e2, e3, e4, e5, e6, e7,
             rows_v, s_sh, dsum_sh, sem_a, sem_b, sem_c):
    cid = lax.axis_index("c")
    sid = lax.axis_index("s")
    wid = sid * NC + cid
    dsts = [d0, d1, d2, d3, d4, d5, d6, d7]
    srcs = [s0, s1, s2, s3, s4, s5, s6, s7]
    dvs = [e0, e1, e2, e3, e4, e5, e6, e7]

    r0 = sid * ROWS_PER_SUB
    pltpu.sync_copy(z16_r.at[pl.ds(r0, ROWS_PER_SUB)],
                    s_sh.at[pl.ds(r0, ROWS_PER_SUB)])
    pltpu.sync_copy(z1_r.at[pl.ds(r0, ROWS_PER_SUB)],
                    dsum_sh.at[pl.ds(r0, ROWS_PER_SUB)])
    plsc.subcore_barrier()

    my_n = jnp.where(wid < N_HI_WORKERS, BLK_HI, BLK_HI - 1)

    def body(t, carry):
        blk = wid + t * NW
        pltpu.sync_copy(ord_r.at[pl.ds(blk * BLK, BLK)], ord_v)
        hs = []
        for j in range(BLK):
            hs.append(pltpu.async_copy(dst_r.at[ord_v.at[j]], dsts[j], sem_a))
            hs.append(pltpu.async_copy(src_r.at[ord_v.at[j]], srcs[j], sem_a))
            hs.append(pltpu.async_copy(dist_r.at[ord_v.at[j]], dvs[j], sem_a))
        for h in hs:
            h.wait()
        hs = []
        for j in range(BLK):
            hs.append(pltpu.async_copy(xp_r.at[srcs[j]], rows_v.at[j], sem_b))
        for h in hs:
            h.wait()
        hs = []
        for j in range(BLK):
            hs.append(pltpu.async_copy(rows_v.at[j], s_sh.at[dsts[j]],
                                       sem_c, add=True))
            hs.append(pltpu.async_copy(dvs[j], dsum_sh.at[dsts[j]],
                                       sem_c, add=True))
        for h in hs:
            h.wait()
        return carry

    lax.fori_loop(0, my_n, body, 0)
    plsc.subcore_barrier()

    pltpu.sync_copy(s_sh.at[pl.ds(r0, ROWS_PER_SUB)],
                    s_out.at[cid, pl.ds(r0, ROWS_PER_SUB)])
    pltpu.sync_copy(dsum_sh.at[pl.ds(r0, ROWS_PER_SUB)],
                    dsum_out.at[cid, pl.ds(r0, ROWS_PER_SUB)])


def _make_sc_kernel():
    mesh = plsc.VectorSubcoreMesh(core_axis_name="c", subcore_axis_name="s",
                                  num_cores=NC, num_subcores=NS)
    idx_bufs = [pltpu.VMEM((IDXW,), jnp.int32) for _ in range(2 * BLK)]
    dv_bufs = [pltpu.VMEM((IDXW,), jnp.float32) for _ in range(BLK)]
    return pl.kernel(
        _sc_body,
        out_type=[
            jax.ShapeDtypeStruct((NC, NPAD, DF), jnp.float32),
            jax.ShapeDtypeStruct((NC, NPAD), jnp.float32),
        ],
        mesh=mesh,
        compiler_params=pltpu.CompilerParams(use_tc_tiling_on_sc=False),
        scratch_types=[
            pltpu.VMEM((BLK, IDXW), jnp.int32),
        ] + idx_bufs + dv_bufs + [
            pltpu.VMEM((BLK, IDXW, DF), jnp.float32),
            pltpu.VMEM_SHARED((NPAD, DF), jnp.float32),
            pltpu.VMEM_SHARED((NPAD,), jnp.float32),
            pltpu.SemaphoreType.DMA,
            pltpu.SemaphoreType.DMA,
            pltpu.SemaphoreType.DMA,
        ],
    )


def _tc_update_body(x_ref, s_ref, ds_ref, w1_ref, wb_ref,
                    bmp_ref, wd_ref, o_ref):
    xb = x_ref[...]
    s = s_ref[0] + s_ref[1]
    ds = ds_ref[0] + ds_ref[1]
    deg = s[:, 10:11]
    t1 = jnp.dot(xb, w1_ref[...], preferred_element_type=jnp.float32)
    t2 = jnp.dot(s, wb_ref[...], preferred_element_type=jnp.float32)
    o_ref[...] = xb + deg * (t1 + bmp_ref[...]) + t2 + ds * wd_ref[...]


def _tc_final_body(x_ref, s_ref, ds_ref, w1_ref, wb_ref,
                   bmp_ref, wd_ref, wo_ref, bo_ref, o_ref):
    xb = x_ref[...]
    s = s_ref[0] + s_ref[1]
    ds = ds_ref[0] + ds_ref[1]
    deg = s[:, 10:11]
    t1 = jnp.dot(xb, w1_ref[...], preferred_element_type=jnp.float32)
    t2 = jnp.dot(s, wb_ref[...], preferred_element_type=jnp.float32)
    xn = xb + deg * (t1 + bmp_ref[...]) + t2 + ds * wd_ref[...]
    o_ref[...] = jnp.dot(xn, wo_ref[...],
                         preferred_element_type=jnp.float32) + bo_ref[...]


def _tc_call(body, dout, xp, s2, ds2, *weights):
    r = ROWS_PER_SUB
    grid = NPAD // r
    wspecs = [pl.BlockSpec(w.shape, lambda i: (0,) * w.ndim) for w in weights]
    return pl.pallas_call(
        body,
        grid=(grid,),
        in_specs=[
            pl.BlockSpec((r, DF), lambda i: (i, 0)),
            pl.BlockSpec((NC, r, DF), lambda i: (0, i, 0)),
            pl.BlockSpec((NC, r, 1), lambda i: (0, i, 0)),
        ] + wspecs,
        out_specs=pl.BlockSpec((r, dout), lambda i: (i, 0)),
        out_shape=jax.ShapeDtypeStruct((NPAD, dout), jnp.float32),
    )(xp, s2, ds2.reshape(NC, NPAD, 1), *weights)


def kernel(x, edge_index, orders, dist, W_mp, b_mp, W_out, b_out):
    f32 = jnp.float32
    dst_tab = edge_index[0].astype(jnp.int32)
    src_tab = edge_index[1].astype(jnp.int32)
    dist_flat = dist.reshape(-1).astype(f32)
    ords = orders.astype(jnp.int32).reshape(2, ORD_ROWS, IDXW)

    wa = W_mp[:10]
    wb = W_mp[10:20]
    w1p = jnp.zeros((DF, DF), f32).at[:10, :10].set(wa - wb)
    wbp = jnp.zeros((DF, DF), f32).at[:10, :10].set(wb)
    bmp = jnp.zeros((1, DF), f32).at[0, :10].set(b_mp)
    wdp = jnp.zeros((1, DF), f32).at[0, :10].set(W_mp[20])
    wop = jnp.zeros((DF, 10), f32).at[:10].set(W_out)
    bo = b_out.reshape(1, 10)

    xp = jnp.zeros((NPAD, DF), f32).at[:N_NODES, :10].set(x.astype(f32))
    xp = xp.at[:, 10].set(1.0)
    z16 = jnp.zeros((NPAD, DF), f32)
    z1 = jnp.zeros((NPAD,), f32)

    sc = _make_sc_kernel()

    s2, ds2 = sc(ords[0], dst_tab, src_tab, dist_flat, xp, z16, z1)
    xp = _tc_call(_tc_update_body, DF, xp, s2, ds2, w1p, wbp, bmp, wdp)
    s2, ds2 = sc(ords[1], dst_tab, src_tab, dist_flat, xp, z16, z1)
    y = _tc_call(_tc_final_body, 10, xp, s2, ds2,
                 w1p, wbp, bmp, wdp, wop, bo)
    return y[:N_NODES]


# speedup vs baseline: 33.7769x; 1.1204x over previous
import jax
import jax.numpy as jnp
from jax import lax
from jax.experimental import pallas as pl
from jax.experimental.pallas import tpu as pltpu
from jax.experimental.pallas import tpu_sc as plsc

N_NODES = 100000
NPAD = 100352
DF = 16
E_ORD = 3200000
IDXW = 128
ORD_ROWS = E_ORD // IDXW
BLK = 4
NBLK = ORD_ROWS // BLK
NC = 2
NS = 16
NW = NC * NS
BLK_HI = -(-NBLK // NW)
N_HI_WORKERS = NBLK - NW * (BLK_HI - 1)
ROWS_PER_SUB = NPAD // NS


def _sc_body(ord_r, dst_r, src_r, dist_r, xp_r, z16_r, z1_r,
             s_out, dsum_out, *scr):
    o = 0
    ordv = scr[o:o + 2]; o += 2
    dsts = [scr[o + BLK * s:o + BLK * (s + 1)] for s in range(2)]
    o += 2 * BLK
    srcs = [scr[o + BLK * s:o + BLK * (s + 1)] for s in range(2)]
    o += 2 * BLK
    dvs = [scr[o + BLK * s:o + BLK * (s + 1)] for s in range(2)]
    o += 2 * BLK
    rows = scr[o:o + 2]; o += 2
    s_sh, dsum_sh = scr[o], scr[o + 1]; o += 2
    sem_o = scr[o:o + 2]
    sem_a = scr[o + 2:o + 4]
    sem_b = scr[o + 4:o + 6]
    sem_c = scr[o + 6:o + 8]

    cid = lax.axis_index("c")
    sid = lax.axis_index("s")
    wid = sid * NC + cid

    r0 = sid * ROWS_PER_SUB
    pltpu.sync_copy(z16_r.at[pl.ds(r0, ROWS_PER_SUB)],
                    s_sh.at[pl.ds(r0, ROWS_PER_SUB)])
    pltpu.sync_copy(z1_r.at[pl.ds(r0, ROWS_PER_SUB)],
                    dsum_sh.at[pl.ds(r0, ROWS_PER_SUB)])
    plsc.subcore_barrier()

    my_n = jnp.where(wid < N_HI_WORKERS, BLK_HI, BLK_HI - 1)
    start = (BLK_HI - 1) * wid + jnp.minimum(wid, N_HI_WORKERS)

    def issue_ord(blk, s):
        return pltpu.async_copy(ord_r.at[pl.ds(blk * BLK, BLK)], ordv[s],
                                sem_o[s])

    def issue_idx(s):
        hs = []
        for j in range(BLK):
            hs.append(pltpu.async_copy(dst_r.at[ordv[s].at[j]], dsts[s][j],
                                       sem_a[s]))
            hs.append(pltpu.async_copy(src_r.at[ordv[s].at[j]], srcs[s][j],
                                       sem_a[s]))
            hs.append(pltpu.async_copy(dist_r.at[ordv[s].at[j]], dvs[s][j],
                                       sem_a[s]))
        return hs

    def issue_rows(s):
        return [pltpu.async_copy(xp_r.at[srcs[s][j]], rows[s].at[j],
                                 sem_b[s]) for j in range(BLK)]

    def issue_scat(s):
        hs = []
        for j in range(BLK):
            hs.append(pltpu.async_copy(rows[s].at[j], s_sh.at[dsts[s][j]],
                                       sem_c[s], add=True))
            hs.append(pltpu.async_copy(dvs[s][j], dsum_sh.at[dsts[s][j]],
                                       sem_c[s], add=True))
        return hs

    def pair(p, carry):
        blk = start + 2 * p
        ho0 = issue_ord(blk, 0)
        ho1 = issue_ord(blk + 1, 1)
        ho0.wait()
        ha0 = issue_idx(0)
        ho1.wait()
        ha1 = issue_idx(1)
        for h in ha0:
            h.wait()
        hb0 = issue_rows(0)
        for h in ha1:
            h.wait()
        hb1 = issue_rows(1)
        for h in hb0:
            h.wait()
        hc0 = issue_scat(0)
        for h in hb1:
            h.wait()
        hc1 = issue_scat(1)
        for h in hc0 + hc1:
            h.wait()
        return carry

    lax.fori_loop(0, my_n // 2, pair, 0)

    @pl.when(my_n % 2 == 1)
    def _tail():
        blk = start + (my_n // 2) * 2
        issue_ord(blk, 0).wait()
        for h in issue_idx(0):
            h.wait()
        for h in issue_rows(0):
            h.wait()
        for h in issue_scat(0):
            h.wait()

    plsc.subcore_barrier()

    pltpu.sync_copy(s_sh.at[pl.ds(r0, ROWS_PER_SUB)],
                    s_out.at[cid, pl.ds(r0, ROWS_PER_SUB)])
    pltpu.sync_copy(dsum_sh.at[pl.ds(r0, ROWS_PER_SUB)],
                    dsum_out.at[cid, pl.ds(r0, ROWS_PER_SUB)])


def _make_sc_kernel():
    mesh = plsc.VectorSubcoreMesh(core_axis_name="c", subcore_axis_name="s",
                                  num_cores=NC, num_subcores=NS)
    i32v = pltpu.VMEM((IDXW,), jnp.int32)
    f32v = pltpu.VMEM((IDXW,), jnp.float32)
    return pl.kernel(
        _sc_body,
        out_type=[
            jax.ShapeDtypeStruct((NC, NPAD, DF), jnp.float32),
            jax.ShapeDtypeStruct((NC, NPAD), jnp.float32),
        ],
        mesh=mesh,
        compiler_params=pltpu.CompilerParams(use_tc_tiling_on_sc=False),
        scratch_types=(
            [pltpu.VMEM((BLK, IDXW), jnp.int32)] * 2
            + [i32v] * (2 * BLK)
            + [i32v] * (2 * BLK)
            + [f32v] * (2 * BLK)
            + [pltpu.VMEM((BLK, IDXW, DF), jnp.float32)] * 2
            + [pltpu.VMEM_SHARED((NPAD, DF), jnp.float32),
               pltpu.VMEM_SHARED((NPAD,), jnp.float32)]
            + [pltpu.SemaphoreType.DMA] * 8
        ),
    )


def _tc_update_body(x_ref, s_ref, ds_ref, w1_ref, wb_ref,
                    bmp_ref, wd_ref, o_ref):
    xb = x_ref[...]
    s = s_ref[0] + s_ref[1]
    ds = ds_ref[0] + ds_ref[1]
    deg = s[:, 10:11]
    t1 = jnp.dot(xb, w1_ref[...], preferred_element_type=jnp.float32)
    t2 = jnp.dot(s, wb_ref[...], preferred_element_type=jnp.float32)
    o_ref[...] = xb + deg * (t1 + bmp_ref[...]) + t2 + ds * wd_ref[...]


def _tc_final_body(x_ref, s_ref, ds_ref, w1_ref, wb_ref,
                   bmp_ref, wd_ref, wo_ref, bo_ref, o_ref):
    xb = x_ref[...]
    s = s_ref[0] + s_ref[1]
    ds = ds_ref[0] + ds_ref[1]
    deg = s[:, 10:11]
    t1 = jnp.dot(xb, w1_ref[...], preferred_element_type=jnp.float32)
    t2 = jnp.dot(s, wb_ref[...], preferred_element_type=jnp.float32)
    xn = xb + deg * (t1 + bmp_ref[...]) + t2 + ds * wd_ref[...]
    o_ref[...] = jnp.dot(xn, wo_ref[...],
                         preferred_element_type=jnp.float32) + bo_ref[...]


def _tc_call(body, dout, xp, s2, ds2, *weights):
    r = ROWS_PER_SUB
    grid = NPAD // r
    wspecs = [pl.BlockSpec(w.shape, lambda i: (0,) * w.ndim) for w in weights]
    return pl.pallas_call(
        body,
        grid=(grid,),
        in_specs=[
            pl.BlockSpec((r, DF), lambda i: (i, 0)),
            pl.BlockSpec((NC, r, DF), lambda i: (0, i, 0)),
            pl.BlockSpec((NC, r, 1), lambda i: (0, i, 0)),
        ] + wspecs,
        out_specs=pl.BlockSpec((r, dout), lambda i: (i, 0)),
        out_shape=jax.ShapeDtypeStruct((NPAD, dout), jnp.float32),
    )(xp, s2, ds2.reshape(NC, NPAD, 1), *weights)


def kernel(x, edge_index, orders, dist, W_mp, b_mp, W_out, b_out):
    f32 = jnp.float32
    dst_tab = edge_index[0].astype(jnp.int32)
    src_tab = edge_index[1].astype(jnp.int32)
    dist_flat = dist.reshape(-1).astype(f32)
    ords = orders.astype(jnp.int32).reshape(2, ORD_ROWS, IDXW)

    wa = W_mp[:10]
    wb = W_mp[10:20]
    w1p = jnp.zeros((DF, DF), f32).at[:10, :10].set(wa - wb)
    wbp = jnp.zeros((DF, DF), f32).at[:10, :10].set(wb)
    bmp = jnp.zeros((1, DF), f32).at[0, :10].set(b_mp)
    wdp = jnp.zeros((1, DF), f32).at[0, :10].set(W_mp[20])
    wop = jnp.zeros((DF, 10), f32).at[:10].set(W_out)
    bo = b_out.reshape(1, 10)

    xp = jnp.zeros((NPAD, DF), f32).at[:N_NODES, :10].set(x.astype(f32))
    xp = xp.at[:, 10].set(1.0)
    z16 = jnp.zeros((NPAD, DF), f32)
    z1 = jnp.zeros((NPAD,), f32)

    sc = _make_sc_kernel()

    s2, ds2 = sc(ords[0], dst_tab, src_tab, dist_flat, xp, z16, z1)
    xp = _tc_call(_tc_update_body, DF, xp, s2, ds2, w1p, wbp, bmp, wdp)
    s2, ds2 = sc(ords[1], dst_tab, src_tab, dist_flat, xp, z16, z1)
    y = _tc_call(_tc_final_body, 10, xp, s2, ds2,
                 w1p, wbp, bmp, wdp, wop, bo)
    return y[:N_NODES]


# speedup vs baseline: 41.6044x; 1.2317x over previous
import jax
import jax.numpy as jnp
from jax import lax
from jax.experimental import pallas as pl
from jax.experimental.pallas import tpu as pltpu
from jax.experimental.pallas import tpu_sc as plsc

N_NODES = 100000
NPAD = 100352
DF = 16
E_ORD = 3200000
IDXW = 128
ORD_ROWS = E_ORD // IDXW
BLK = 4
NBLK = ORD_ROWS // BLK
NC = 2
NS = 16
NW = NC * NS
BLK_HI = -(-NBLK // NW)
N_HI_WORKERS = NBLK - NW * (BLK_HI - 1)
ROWS_PER_SUB = NPAD // NS


def _sc_body(ord_r, dst_r, src_r, dist_r, xp_r, z16_r, z1_r,
             s_out, dsum_out, *scr):
    o = 0
    ordv = scr[o:o + 2]; o += 2
    dsts = [scr[o + BLK * s:o + BLK * (s + 1)] for s in range(2)]
    o += 2 * BLK
    srcs = [scr[o + BLK * s:o + BLK * (s + 1)] for s in range(2)]
    o += 2 * BLK
    dvs = [scr[o + BLK * s:o + BLK * (s + 1)] for s in range(2)]
    o += 2 * BLK
    rows = scr[o:o + 2]; o += 2
    s_sh, dsum_sh = scr[o], scr[o + 1]; o += 2
    sem_o = scr[o:o + 2]
    sem_a = scr[o + 2:o + 4]
    sem_b = scr[o + 4:o + 6]
    sem_c = scr[o + 6:o + 8]

    cid = lax.axis_index("c")
    sid = lax.axis_index("s")
    wid = sid * NC + cid

    r0 = sid * ROWS_PER_SUB
    pltpu.sync_copy(z16_r.at[pl.ds(r0, ROWS_PER_SUB)],
                    s_sh.at[pl.ds(r0, ROWS_PER_SUB)])
    pltpu.sync_copy(z1_r.at[pl.ds(r0, ROWS_PER_SUB)],
                    dsum_sh.at[pl.ds(r0, ROWS_PER_SUB)])
    plsc.subcore_barrier()

    my_n = jnp.where(wid < N_HI_WORKERS, BLK_HI, BLK_HI - 1)
    start = (BLK_HI - 1) * wid + jnp.minimum(wid, N_HI_WORKERS)

    def issue_ord(blk, s):
        return pltpu.async_copy(ord_r.at[pl.ds(blk * BLK, BLK)], ordv[s],
                                sem_o[s])

    def issue_idx(s):
        hs = []
        for j in range(BLK):
            hs.append(pltpu.async_copy(dst_r.at[ordv[s].at[j]], dsts[s][j],
                                       sem_a[s]))
            hs.append(pltpu.async_copy(src_r.at[ordv[s].at[j]], srcs[s][j],
                                       sem_a[s]))
            hs.append(pltpu.async_copy(dist_r.at[ordv[s].at[j]], dvs[s][j],
                                       sem_a[s]))
        return hs

    def issue_rows(s):
        return [pltpu.async_copy(xp_r.at[srcs[s][j]], rows[s].at[j],
                                 sem_b[s]) for j in range(BLK)]

    def issue_scat(s):
        hs = []
        for j in range(BLK):
            hs.append(pltpu.async_copy(rows[s].at[j], s_sh.at[dsts[s][j]],
                                       sem_c[s], add=True))
            hs.append(pltpu.async_copy(dvs[s][j], dsum_sh.at[dsts[s][j]],
                                       sem_c[s], add=True))
        return hs

    def pair(p, carry):
        blk = start + 2 * p
        ho0 = issue_ord(blk, 0)
        ho1 = issue_ord(blk + 1, 1)
        ho0.wait()
        ha0 = issue_idx(0)
        ho1.wait()
        ha1 = issue_idx(1)
        for h in ha0:
            h.wait()
        hb0 = issue_rows(0)
        for h in ha1:
            h.wait()
        hb1 = issue_rows(1)
        for h in hb0:
            h.wait()
        hc0 = issue_scat(0)
        for h in hb1:
            h.wait()
        hc1 = issue_scat(1)
        for h in hc0 + hc1:
            h.wait()
        return carry

    lax.fori_loop(0, my_n // 2, pair, 0)

    @pl.when(my_n % 2 == 1)
    def _tail():
        blk = start + (my_n // 2) * 2
        issue_ord(blk, 0).wait()
        for h in issue_idx(0):
            h.wait()
        for h in issue_rows(0):
            h.wait()
        for h in issue_scat(0):
            h.wait()

    plsc.subcore_barrier()

    pltpu.sync_copy(s_sh.at[pl.ds(r0, ROWS_PER_SUB)],
                    s_out.at[cid, pl.ds(r0, ROWS_PER_SUB)])
    pltpu.sync_copy(dsum_sh.at[pl.ds(r0, ROWS_PER_SUB)],
                    dsum_out.at[cid, pl.ds(r0, ROWS_PER_SUB)])


def _make_sc_kernel():
    mesh = plsc.VectorSubcoreMesh(core_axis_name="c", subcore_axis_name="s",
                                  num_cores=NC, num_subcores=NS)
    i32v = pltpu.VMEM((IDXW,), jnp.int32)
    f32v = pltpu.VMEM((IDXW,), jnp.float32)
    return pl.kernel(
        _sc_body,
        out_type=[
            jax.ShapeDtypeStruct((NC, NPAD, DF), jnp.float32),
            jax.ShapeDtypeStruct((NC, NPAD), jnp.float32),
        ],
        mesh=mesh,
        compiler_params=pltpu.CompilerParams(use_tc_tiling_on_sc=False),
        scratch_types=(
            [pltpu.VMEM((BLK, IDXW), jnp.int32)] * 2
            + [i32v] * (2 * BLK)
            + [i32v] * (2 * BLK)
            + [f32v] * (2 * BLK)
            + [pltpu.VMEM((BLK, IDXW, DF), jnp.float32)] * 2
            + [pltpu.VMEM_SHARED((NPAD, DF), jnp.float32),
               pltpu.VMEM_SHARED((NPAD,), jnp.float32)]
            + [pltpu.SemaphoreType.DMA] * 8
        ),
    )


NR = NPAD * DF // 128


def _tc_update_body(x_ref, s_ref, ds_ref, w1_ref, wb_ref, md_ref,
                    bmp_ref, wd_ref, o_ref):
    xb = x_ref[...]
    s = s_ref[0] + s_ref[1]
    ds = ds_ref[...]
    deg = jnp.dot(s, md_ref[...], preferred_element_type=jnp.float32)
    t1 = jnp.dot(xb, w1_ref[...], preferred_element_type=jnp.float32)
    t2 = jnp.dot(s, wb_ref[...], preferred_element_type=jnp.float32)
    o_ref[...] = xb + deg * (t1 + bmp_ref[...]) + t2 + ds * wd_ref[...]


def _tc_final_body(x_ref, s_ref, ds_ref, w1_ref, wb_ref, md_ref,
                   bmp_ref, wd_ref, wo_ref, bo_ref, o_ref):
    xb = x_ref[...]
    s = s_ref[0] + s_ref[1]
    ds = ds_ref[...]
    deg = jnp.dot(s, md_ref[...], preferred_element_type=jnp.float32)
    t1 = jnp.dot(xb, w1_ref[...], preferred_element_type=jnp.float32)
    t2 = jnp.dot(s, wb_ref[...], preferred_element_type=jnp.float32)
    xn = xb + deg * (t1 + bmp_ref[...]) + t2 + ds * wd_ref[...]
    o_ref[...] = jnp.dot(xn, wo_ref[...],
                         preferred_element_type=jnp.float32) + bo_ref[...]


def _tc_call(body, xp, s2, ds2, *weights):
    r = NR // 16
    grid = NR // r
    x128 = xp.reshape(NR, 128)
    s128 = s2.reshape(NC, NR, 128)
    ds128 = jnp.repeat(ds2[0] + ds2[1], DF).reshape(NR, 128)
    wspecs = [pl.BlockSpec(w.shape, lambda i: (0,) * w.ndim) for w in weights]
    out = pl.pallas_call(
        body,
        grid=(grid,),
        in_specs=[
            pl.BlockSpec((r, 128), lambda i: (i, 0)),
            pl.BlockSpec((NC, r, 128), lambda i: (0, i, 0)),
            pl.BlockSpec((r, 128), lambda i: (i, 0)),
        ] + wspecs,
        out_specs=pl.BlockSpec((r, 128), lambda i: (i, 0)),
        out_shape=jax.ShapeDtypeStruct((NR, 128), jnp.float32),
    )(x128, s128, ds128, *weights)
    return out.reshape(NPAD, DF)


def kernel(x, edge_index, orders, dist, W_mp, b_mp, W_out, b_out):
    f32 = jnp.float32
    dst_tab = edge_index[0].astype(jnp.int32)
    src_tab = edge_index[1].astype(jnp.int32)
    dist_flat = dist.reshape(-1).astype(f32)
    ords = orders.astype(jnp.int32).reshape(2, ORD_ROWS, IDXW)

    wa = W_mp[:10]
    wb = W_mp[10:20]
    w1p = jnp.zeros((DF, DF), f32).at[:10, :10].set(wa - wb)
    wbp = jnp.zeros((DF, DF), f32).at[:10, :10].set(wb)
    wop = jnp.zeros((DF, DF), f32).at[:10, :10].set(W_out)
    msel = jnp.zeros((DF, DF), f32).at[10, :].set(1.0)
    eye8 = jnp.eye(8, dtype=f32)
    w1B = jnp.kron(eye8, w1p)
    wbB = jnp.kron(eye8, wbp)
    woB = jnp.kron(eye8, wop)
    mdB = jnp.kron(eye8, msel)
    bmpB = jnp.tile(jnp.zeros((1, DF), f32).at[0, :10].set(b_mp), (1, 8))
    wdB = jnp.tile(jnp.zeros((1, DF), f32).at[0, :10].set(W_mp[20]), (1, 8))
    boB = jnp.tile(jnp.zeros((1, DF), f32).at[0, :10].set(b_out), (1, 8))

    xp = jnp.zeros((NPAD, DF), f32).at[:N_NODES, :10].set(x.astype(f32))
    xp = xp.at[:, 10].set(1.0)
    z16 = jnp.zeros((NPAD, DF), f32)
    z1 = jnp.zeros((NPAD,), f32)

    sc = _make_sc_kernel()

    s2, ds2 = sc(ords[0], dst_tab, src_tab, dist_flat, xp, z16, z1)
    xp = _tc_call(_tc_update_body, xp, s2, ds2, w1B, wbB, mdB, bmpB, wdB)
    s2, ds2 = sc(ords[1], dst_tab, src_tab, dist_flat, xp, z16, z1)
    y = _tc_call(_tc_final_body, xp, s2, ds2,
                 w1B, wbB, mdB, bmpB, wdB, woB, boB)
    return y[:N_NODES, :10]


# speedup vs baseline: 46.3922x; 1.1151x over previous
import jax
import jax.numpy as jnp
from jax import lax
from jax.experimental import pallas as pl
from jax.experimental.pallas import tpu as pltpu
from jax.experimental.pallas import tpu_sc as plsc

N_NODES = 100000
NPAD = 100352
DF = 16
E_ORD = 3200000
IDXW = 128
ORD_ROWS = E_ORD // IDXW
BLK = 4
NBLK = ORD_ROWS // BLK
NC = 2
NS = 16
NW = NC * NS
BLK_HI = -(-NBLK // NW)
N_HI_WORKERS = NBLK - NW * (BLK_HI - 1)
ROWS_PER_SUB = NPAD // NS


def _sc_body(ord_r, dst_r, src_r, dist_r, xp_r, z16_r, z1_r,
             s_out, dsum_out, *scr):
    o = 0
    ordv = scr[o:o + 2]; o += 2
    dsts = [scr[o + BLK * s:o + BLK * (s + 1)] for s in range(2)]
    o += 2 * BLK
    srcs = [scr[o + BLK * s:o + BLK * (s + 1)] for s in range(2)]
    o += 2 * BLK
    dvs = [scr[o + BLK * s:o + BLK * (s + 1)] for s in range(2)]
    o += 2 * BLK
    rows = scr[o:o + 2]; o += 2
    s_sh, dsum_sh = scr[o], scr[o + 1]; o += 2
    sem_o = scr[o:o + 2]
    sem_a = scr[o + 2:o + 4]
    sem_b = scr[o + 4:o + 6]
    sem_c = scr[o + 6:o + 8]

    cid = lax.axis_index("c")
    sid = lax.axis_index("s")
    wid = sid * NC + cid

    r0 = sid * ROWS_PER_SUB
    pltpu.sync_copy(z16_r.at[pl.ds(r0, ROWS_PER_SUB)],
                    s_sh.at[pl.ds(r0, ROWS_PER_SUB)])
    pltpu.sync_copy(z1_r.at[pl.ds(r0, ROWS_PER_SUB)],
                    dsum_sh.at[pl.ds(r0, ROWS_PER_SUB)])
    plsc.subcore_barrier()

    my_n = jnp.where(wid < N_HI_WORKERS, BLK_HI, BLK_HI - 1)
    start = (BLK_HI - 1) * wid + jnp.minimum(wid, N_HI_WORKERS)

    def issue_ord(blk, s):
        return pltpu.async_copy(ord_r.at[pl.ds(blk * BLK, BLK)], ordv[s],
                                sem_o[s])

    def issue_idx(s):
        hs = []
        for j in range(BLK):
            hs.append(pltpu.async_copy(dst_r.at[ordv[s].at[j]], dsts[s][j],
                                       sem_a[s]))
            hs.append(pltpu.async_copy(src_r.at[ordv[s].at[j]], srcs[s][j],
                                       sem_a[s]))
            hs.append(pltpu.async_copy(dist_r.at[ordv[s].at[j]], dvs[s][j],
                                       sem_a[s]))
        return hs

    def issue_rows(s):
        return [pltpu.async_copy(xp_r.at[srcs[s][j]], rows[s].at[j],
                                 sem_b[s]) for j in range(BLK)]

    def issue_scat(s):
        hs = []
        for j in range(BLK):
            hs.append(pltpu.async_copy(rows[s].at[j], s_sh.at[dsts[s][j]],
                                       sem_c[s], add=True))
            hs.append(pltpu.async_copy(dvs[s][j], dsum_sh.at[dsts[s][j]],
                                       sem_c[s], add=True))
        return hs

    def pair(p, carry):
        blk = start + 2 * p
        ho0 = issue_ord(blk, 0)
        ho1 = issue_ord(blk + 1, 1)
        ho0.wait()
        ha0 = issue_idx(0)
        ho1.wait()
        ha1 = issue_idx(1)
        for h in ha0:
            h.wait()
        hb0 = issue_rows(0)
        for h in ha1:
            h.wait()
        hb1 = issue_rows(1)
        for h in hb0:
            h.wait()
        hc0 = issue_scat(0)
        for h in hb1:
            h.wait()
        hc1 = issue_scat(1)
        for h in hc0 + hc1:
            h.wait()
        return carry

    lax.fori_loop(0, my_n // 2, pair, 0)

    @pl.when(my_n % 2 == 1)
    def _tail():
        blk = start + (my_n // 2) * 2
        issue_ord(blk, 0).wait()
        for h in issue_idx(0):
            h.wait()
        for h in issue_rows(0):
            h.wait()
        for h in issue_scat(0):
            h.wait()

    plsc.subcore_barrier()

    pltpu.sync_copy(s_sh.at[pl.ds(r0, ROWS_PER_SUB)],
                    s_out.at[cid, pl.ds(r0, ROWS_PER_SUB)])
    pltpu.sync_copy(dsum_sh.at[pl.ds(r0, ROWS_PER_SUB)],
                    dsum_out.at[cid, pl.ds(r0, ROWS_PER_SUB)])


def _make_sc_kernel():
    mesh = plsc.VectorSubcoreMesh(core_axis_name="c", subcore_axis_name="s",
                                  num_cores=NC, num_subcores=NS)
    i32v = pltpu.VMEM((IDXW,), jnp.int32)
    f32v = pltpu.VMEM((IDXW,), jnp.float32)
    return pl.kernel(
        _sc_body,
        out_type=[
            jax.ShapeDtypeStruct((NC, NPAD, DF), jnp.float32),
            jax.ShapeDtypeStruct((NC, NPAD), jnp.float32),
        ],
        mesh=mesh,
        compiler_params=pltpu.CompilerParams(use_tc_tiling_on_sc=False),
        scratch_types=(
            [pltpu.VMEM((BLK, IDXW), jnp.int32)] * 2
            + [i32v] * (2 * BLK)
            + [i32v] * (2 * BLK)
            + [f32v] * (2 * BLK)
            + [pltpu.VMEM((BLK, IDXW, DF), jnp.float32)] * 2
            + [pltpu.VMEM_SHARED((NPAD, DF), jnp.float32),
               pltpu.VMEM_SHARED((NPAD,), jnp.float32)]
            + [pltpu.SemaphoreType.DMA] * 8
        ),
    )


NR = NPAD * DF // 128


def _tc_update_body(x_ref, s_ref, ds_ref, w1_ref, wb_ref, md_ref,
                    bmp_ref, wd_ref, o_ref):
    xb = x_ref[...]
    s = s_ref[0] + s_ref[1]
    ds = ds_ref[...]
    deg = jnp.dot(s, md_ref[...], preferred_element_type=jnp.float32)
    t1 = jnp.dot(xb, w1_ref[...], preferred_element_type=jnp.float32)
    t2 = jnp.dot(s, wb_ref[...], preferred_element_type=jnp.float32)
    o_ref[...] = xb + deg * (t1 + bmp_ref[...]) + t2 + ds * wd_ref[...]


def _tc_final_body(x_ref, s_ref, ds_ref, w1_ref, wb_ref, md_ref,
                   bmp_ref, wd_ref, wo_ref, bo_ref, o_ref):
    xb = x_ref[...]
    s = s_ref[0] + s_ref[1]
    ds = ds_ref[...]
    deg = jnp.dot(s, md_ref[...], preferred_element_type=jnp.float32)
    t1 = jnp.dot(xb, w1_ref[...], preferred_element_type=jnp.float32)
    t2 = jnp.dot(s, wb_ref[...], preferred_element_type=jnp.float32)
    xn = xb + deg * (t1 + bmp_ref[...]) + t2 + ds * wd_ref[...]
    o_ref[...] = jnp.dot(xn, wo_ref[...],
                         preferred_element_type=jnp.float32) + bo_ref[...]


def _tc_call(body, xp, s2, ds2, *weights):
    r = NR // 16
    grid = NR // r
    x128 = xp.reshape(NR, 128)
    s128 = s2.reshape(NC, NR, 128)
    ds128 = jnp.repeat(ds2[0] + ds2[1], DF).reshape(NR, 128)
    wspecs = [pl.BlockSpec(w.shape, lambda i: (0,) * w.ndim) for w in weights]
    out = pl.pallas_call(
        body,
        grid=(grid,),
        in_specs=[
            pl.BlockSpec((r, 128), lambda i: (i, 0)),
            pl.BlockSpec((NC, r, 128), lambda i: (0, i, 0)),
            pl.BlockSpec((r, 128), lambda i: (i, 0)),
        ] + wspecs,
        out_specs=pl.BlockSpec((r, 128), lambda i: (i, 0)),
        out_shape=jax.ShapeDtypeStruct((NR, 128), jnp.float32),
    )(x128, s128, ds128, *weights)
    return out.reshape(NPAD, DF)


def kernel(x, edge_index, orders, dist, W_mp, b_mp, W_out, b_out):
    f32 = jnp.float32
    dst_tab = edge_index[0].astype(jnp.int32)
    src_tab = edge_index[1].astype(jnp.int32)
    dist_flat = dist.reshape(-1).astype(f32)
    ords = orders.astype(jnp.int32).reshape(2, ORD_ROWS, IDXW)

    wa = W_mp[:10]
    wb = W_mp[10:20]
    w1p = jnp.zeros((DF, DF), f32).at[:10, :10].set(wa - wb)
    wbp = jnp.zeros((DF, DF), f32).at[:10, :10].set(wb)
    wop = jnp.zeros((DF, DF), f32).at[:10, :10].set(W_out)
    msel = jnp.zeros((DF, DF), f32).at[10, :].set(1.0)
    eye8 = jnp.eye(8, dtype=f32)
    w1B = jnp.kron(eye8, w1p)
    wbB = jnp.kron(eye8, wbp)
    woB = jnp.kron(eye8, wop)
    mdB = jnp.kron(eye8, msel)
    bmpB = jnp.tile(jnp.zeros((1, DF), f32).at[0, :10].set(b_mp), (1, 8))
    wdB = jnp.tile(jnp.zeros((1, DF), f32).at[0, :10].set(W_mp[20]), (1, 8))
    boB = jnp.tile(jnp.zeros((1, DF), f32).at[0, :10].set(b_out), (1, 8))

    xw = jnp.concatenate(
        [x.astype(f32), jnp.ones((N_NODES, 1), f32),
         jnp.zeros((N_NODES, DF - 11), f32)], axis=1)
    xp = jnp.pad(xw, ((0, NPAD - N_NODES), (0, 0)))
    z16 = jnp.zeros((NPAD, DF), f32)
    z1 = jnp.zeros((NPAD,), f32)

    sc = _make_sc_kernel()

    s2, ds2 = sc(ords[0], dst_tab, src_tab, dist_flat, xp, z16, z1)
    xp = _tc_call(_tc_update_body, xp, s2, ds2, w1B, wbB, mdB, bmpB, wdB)
    s2, ds2 = sc(ords[1], dst_tab, src_tab, dist_flat, xp, z16, z1)
    y = _tc_call(_tc_final_body, xp, s2, ds2,
                 w1B, wbB, mdB, bmpB, wdB, woB, boB)
    return y[:N_NODES, :10]
